# baseline (speedup 1.0000x reference)
import jax
import jax.numpy as jnp
from jax import lax
from jax.experimental import pallas as pl
from jax.experimental.pallas import tpu as pltpu
from jax.experimental.pallas import tpu_sc as plsc

B, S, D = 4096, 77, 128
LANES = 16
CH = 64
NB = 7
GA = 2
RU = 8

_info = plsc.get_sparse_core_info()
NC, NS = _info.num_cores, _info.num_subcores
NW = NC * NS
TOK_PER_W = B * S // NW
NITER = TOK_PER_W // CH
POSX = S + CH


def _emb_body(ids_hbm, w_hbm, pos_hbm, out_hbm, idx_v, rows_v, posx_v, gsem, ssem):
    wid = lax.axis_index("s") * NC + lax.axis_index("c")
    tok0 = wid * TOK_PER_W
    pltpu.sync_copy(pos_hbm, posx_v.at[pl.ds(0, S)])
    pltpu.sync_copy(pos_hbm.at[pl.ds(0, POSX - S)], posx_v.at[pl.ds(S, POSX - S)])
    pltpu.sync_copy(ids_hbm.at[pl.ds(tok0, TOK_PER_W)], idx_v)

    def g_copy(ci, b):
        return pltpu.make_async_copy(
            w_hbm.at[idx_v.at[pl.ds(ci * CH, CH)]], rows_v.at[b], gsem.at[b])

    def s_copy(ci, b):
        return pltpu.make_async_copy(
            rows_v.at[b], out_hbm.at[pl.ds(tok0 + ci * CH, CH), :], ssem.at[b])

    for a in range(GA):
        g_copy(a, a).start()

    def outer(g, carry):
        for b in range(NB):
            ci = g * NB + b
            nb = (b + GA) % NB

            @pl.when(ci + GA < NITER)
            def _():
                @pl.when(ci >= NB - GA)
                def _():
                    s_copy(ci + GA - NB, nb).wait()
                g_copy(ci + GA, nb).start()

            g_copy(ci, b).wait()
            start = lax.rem(ci * CH, S)

            def row(rr, c2):
                for k in range(RU):
                    r = rr * RU + k
                    pr = start + r
                    for c in range(D // LANES):
                        sl = pl.ds(c * LANES, LANES)
                        rows_v[b, r, sl] = rows_v[b, r, sl] + posx_v[pr, sl]
                return c2

            lax.fori_loop(0, CH // RU, row, 0)
            s_copy(ci, b).start()
        return carry

    lax.fori_loop(0, NITER // NB, outer, 0)
    for b in range(NB):
        s_copy(NITER - NB + b, b).wait()


@jax.jit
def kernel(input_ids, weight, position_embedding):
    mesh = plsc.VectorSubcoreMesh(core_axis_name="c", subcore_axis_name="s")
    out = pl.kernel(
        _emb_body,
        mesh=mesh,
        out_type=jax.ShapeDtypeStruct((B * S, D), jnp.float32),
        scratch_types=[
            pltpu.VMEM((TOK_PER_W,), jnp.int32),
            pltpu.VMEM((NB, CH, D), jnp.float32),
            pltpu.VMEM((POSX, D), jnp.float32),
            pltpu.SemaphoreType.DMA((NB,)),
            pltpu.SemaphoreType.DMA((NB,)),
        ],
    )(input_ids.reshape(B * S).astype(jnp.int32), weight, position_embedding)
    return out.reshape(B, S, D)


# speedup vs baseline: 1.5906x; 1.5906x over previous
import jax
import jax.numpy as jnp
from jax import lax
from jax.experimental import pallas as pl
from jax.experimental.pallas import tpu as pltpu
from jax.experimental.pallas import tpu_sc as plsc

B, S, D = 4096, 77, 128
LANES = 16
NB = 8
GA = 2
RU = 7

K = 4
BK = B // K

_info = plsc.get_sparse_core_info()
NC, NS = _info.num_cores, _info.num_subcores
NW = NC * NS
SEQ_PER_W = BK // NW


def _emb_body(ids_hbm, w_hbm, pos_hbm, out_hbm, idx_v, rows_v, pos_v, gsem, ssem):
    wid = lax.axis_index("s") * NC + lax.axis_index("c")
    seq0 = wid * SEQ_PER_W
    pltpu.sync_copy(pos_hbm, pos_v)
    pltpu.sync_copy(ids_hbm.at[pl.ds(seq0, SEQ_PER_W), :], idx_v)

    def g_copy(ci, b):
        return pltpu.make_async_copy(
            w_hbm.at[idx_v.at[ci]], rows_v.at[b], gsem.at[b])

    def s_copy(ci, b):
        return pltpu.make_async_copy(
            rows_v.at[b], out_hbm.at[seq0 + ci], ssem.at[b])

    for a in range(GA):
        g_copy(a, a).start()

    def outer(g, carry):
        for b in range(NB):
            ci = g * NB + b
            nb = (b + GA) % NB

            @pl.when(ci + GA < SEQ_PER_W)
            def _():
                @pl.when(ci >= NB - GA)
                def _():
                    s_copy(ci + GA - NB, nb).wait()
                g_copy(ci + GA, nb).start()

            g_copy(ci, b).wait()

            def row(rr, c2):
                for k in range(RU):
                    r = rr * RU + k
                    for c in range(D // LANES):
                        sl = pl.ds(c * LANES, LANES)
                        rows_v[b, r, sl] = rows_v[b, r, sl] + pos_v[r, sl]
                return c2

            lax.fori_loop(0, S // RU, row, 0)
            s_copy(ci, b).start()
        return carry

    lax.fori_loop(0, SEQ_PER_W // NB, outer, 0)
    for b in range(NB):
        s_copy(SEQ_PER_W - NB + b, b).wait()


@jax.jit
def kernel(input_ids, weight, position_embedding):
    mesh = plsc.VectorSubcoreMesh(core_axis_name="c", subcore_axis_name="s")
    call = pl.kernel(
        _emb_body,
        mesh=mesh,
        out_type=jax.ShapeDtypeStruct((BK, S, D), jnp.float32),
        scratch_types=[
            pltpu.VMEM((SEQ_PER_W, S), jnp.int32),
            pltpu.VMEM((NB, S, D), jnp.float32),
            pltpu.VMEM((S, D), jnp.float32),
            pltpu.SemaphoreType.DMA((NB,)),
            pltpu.SemaphoreType.DMA((NB,)),
        ],
    )
    ids = input_ids.astype(jnp.int32)
    outs = [
        call(lax.slice_in_dim(ids, k * BK, (k + 1) * BK, axis=0),
             weight, position_embedding)
        for k in range(K)
    ]
    return jnp.concatenate(outs, axis=0)


# speedup vs baseline: 4.7018x; 2.9560x over previous
import jax
import jax.numpy as jnp
from jax import lax
from jax.experimental import pallas as pl
from jax.experimental.pallas import tpu as pltpu
from jax.experimental.pallas import tpu_sc as plsc

B, S, D = 4096, 77, 128
LANES = 16
NB = 8
GA = 2
RU = 7

_info = plsc.get_sparse_core_info()
NC, NS = _info.num_cores, _info.num_subcores
NW = NC * NS
SEQ_PER_W = B // NW

_FILL_OFFS = (0, 16, 32, 48, S - LANES)


def _emb_body(ids_hbm, w_hbm, pos_hbm, base_hbm, out_hbm,
              idx_v, rows_v, pos_v, base_v, sidx_v, gsem, ssem):
    wid = lax.axis_index("s") * NC + lax.axis_index("c")
    seq0 = wid * SEQ_PER_W
    pltpu.sync_copy(pos_hbm, pos_v)
    pltpu.sync_copy(base_hbm, base_v)
    pltpu.sync_copy(ids_hbm.at[pl.ds(seq0, SEQ_PER_W), :], idx_v)

    def g_copy(ci, b):
        return pltpu.make_async_copy(
            w_hbm.at[idx_v.at[ci]], rows_v.at[b], gsem.at[b])

    def s_copy(b):
        return pltpu.make_async_copy(
            rows_v.at[b], out_hbm.at[sidx_v.at[b]], ssem.at[b])

    for a in range(GA):
        g_copy(a, a).start()

    def outer(g, carry):
        for b in range(NB):
            ci = g * NB + b
            nb = (b + GA) % NB

            @pl.when(ci + GA < SEQ_PER_W)
            def _():
                @pl.when(ci >= NB - GA)
                def _():
                    s_copy(nb).wait()
                g_copy(ci + GA, nb).start()

            g_copy(ci, b).wait()

            def row(rr, c2):
                for k in range(RU):
                    r = rr * RU + k
                    for c in range(D // LANES):
                        sl = pl.ds(c * LANES, LANES)
                        rows_v[b, r, sl] = rows_v[b, r, sl] + pos_v[r, sl]
                return c2

            lax.fori_loop(0, S // RU, row, 0)
            for off in _FILL_OFFS:
                sl = pl.ds(off, LANES)
                sidx_v[b, sl] = base_v[sl] + (seq0 + ci)
            s_copy(b).start()
        return carry

    lax.fori_loop(0, SEQ_PER_W // NB, outer, 0)
    for b in range(NB):
        s_copy(b).wait()


@jax.jit
def kernel(input_ids, weight, position_embedding):
    mesh = plsc.VectorSubcoreMesh(core_axis_name="c", subcore_axis_name="s")
    base = jnp.arange(S, dtype=jnp.int32) * B
    out = pl.kernel(
        _emb_body,
        mesh=mesh,
        out_type=jax.ShapeDtypeStruct((S * B, D), jnp.float32),
        scratch_types=[
            pltpu.VMEM((SEQ_PER_W, S), jnp.int32),
            pltpu.VMEM((NB, S, D), jnp.float32),
            pltpu.VMEM((S, D), jnp.float32),
            pltpu.VMEM((S,), jnp.int32),
            pltpu.VMEM((NB, S), jnp.int32),
            pltpu.SemaphoreType.DMA((NB,)),
            pltpu.SemaphoreType.DMA((NB,)),
        ],
    )(input_ids.astype(jnp.int32), weight, position_embedding, base)
    return out.reshape(S, B, D).transpose(1, 0, 2)


# speedup vs baseline: 4.9785x; 1.0589x over previous
import jax
import jax.numpy as jnp
from jax import lax
from jax.experimental import pallas as pl
from jax.experimental.pallas import tpu as pltpu
from jax.experimental.pallas import tpu_sc as plsc

B, S, D = 4096, 77, 128
LANES = 16
P = 2
NB = 4
GA = 2
RU = 7

_info = plsc.get_sparse_core_info()
NC, NS = _info.num_cores, _info.num_subcores
NW = NC * NS
SEQ_PER_W = B // NW
CHUNKS = SEQ_PER_W // P

_FILL_OFFS = (0, 16, 32, 48, S - LANES)


def _emb_body(ids_hbm, w_hbm, pos_hbm, base_hbm, out_hbm,
              idx_v, rows_v, pos_v, base_v, sidx_v, gsem, ssem):
    wid = lax.axis_index("s") * NC + lax.axis_index("c")
    seq0 = wid * SEQ_PER_W
    pltpu.sync_copy(pos_hbm, pos_v)
    pltpu.sync_copy(base_hbm, base_v)
    pltpu.sync_copy(ids_hbm.at[pl.ds(seq0, SEQ_PER_W), :], idx_v)

    def g_copy(ci, b, j):
        return pltpu.make_async_copy(
            w_hbm.at[idx_v.at[ci * P + j]],
            rows_v.at[b, pl.ds(j * S, S)], gsem.at[b])

    def s_copy(b, j):
        return pltpu.make_async_copy(
            rows_v.at[b, pl.ds(j * S, S)],
            out_hbm.at[sidx_v.at[b, j]], ssem.at[b])

    for a in range(GA):
        for j in range(P):
            g_copy(a, a, j).start()

    def outer(g, carry):
        for b in range(NB):
            ci = g * NB + b
            nb = (b + GA) % NB

            @pl.when(ci + GA < CHUNKS)
            def _():
                @pl.when(ci >= NB - GA)
                def _():
                    for j in range(P):
                        s_copy(nb, j).wait()
                for j in range(P):
                    g_copy(ci + GA, nb, j).start()

            for j in range(P):
                g_copy(ci, b, j).wait()

            def row(rr, c2):
                for k in range(RU):
                    r = rr * RU + k
                    for c in range(D // LANES):
                        sl = pl.ds(c * LANES, LANES)
                        pv = pos_v[r, sl]
                        rows_v[b, r, sl] = rows_v[b, r, sl] + pv
                        rows_v[b, S + r, sl] = rows_v[b, S + r, sl] + pv
                return c2

            lax.fori_loop(0, S // RU, row, 0)
            for j in range(P):
                for off in _FILL_OFFS:
                    sl = pl.ds(off, LANES)
                    sidx_v[b, j, sl] = base_v[sl] + (seq0 + ci * P + j)
                s_copy(b, j).start()
        return carry

    lax.fori_loop(0, CHUNKS // NB, outer, 0)
    for b in range(NB):
        for j in range(P):
            s_copy(b, j).wait()


@jax.jit
def kernel(input_ids, weight, position_embedding):
    mesh = plsc.VectorSubcoreMesh(core_axis_name="c", subcore_axis_name="s")
    base = jnp.arange(S, dtype=jnp.int32) * B
    out = pl.kernel(
        _emb_body,
        mesh=mesh,
        out_type=jax.ShapeDtypeStruct((S * B, D), jnp.float32),
        scratch_types=[
            pltpu.VMEM((SEQ_PER_W, S), jnp.int32),
            pltpu.VMEM((NB, P * S, D), jnp.float32),
            pltpu.VMEM((S, D), jnp.float32),
            pltpu.VMEM((S,), jnp.int32),
            pltpu.VMEM((NB, P, S), jnp.int32),
            pltpu.SemaphoreType.DMA((NB,)),
            pltpu.SemaphoreType.DMA((NB,)),
        ],
    )(input_ids.astype(jnp.int32), weight, position_embedding, base)
    return out.reshape(S, B, D).transpose(1, 0, 2)


# speedup vs baseline: 5.2137x; 1.0472x over previous
import jax
import jax.numpy as jnp
from jax import lax
from jax.experimental import pallas as pl
from jax.experimental.pallas import tpu as pltpu
from jax.experimental.pallas import tpu_sc as plsc

B, S, D = 4096, 77, 128
LANES = 16
CH = 128
NB = 7
GA = 2
RU = 8
PW = 16

_info = plsc.get_sparse_core_info()
NC, NS = _info.num_cores, _info.num_subcores
NW = NC * NS
ROWS_PER_W = S * B // NW
CHUNKS = ROWS_PER_W // CH
CPP = B // CH


def _emb_body(ids_hbm, w_hbm, pos_hbm, out_hbm, idx_v, rows_v, pos_v, gsem, ssem):
    wid = lax.axis_index("s") * NC + lax.axis_index("c")
    cc0 = wid * CHUNKS
    p_lo = cc0 // CPP
    ab = pl.multiple_of((p_lo // 8) * 8, 8)
    pltpu.sync_copy(pos_hbm.at[pl.ds(ab, PW), :], pos_v)
    pltpu.sync_copy(ids_hbm.at[wid], idx_v)

    def g_copy(ci, b):
        return pltpu.make_async_copy(
            w_hbm.at[idx_v.at[ci]], rows_v.at[b], gsem.at[b])

    def s_copy(ci, b):
        return pltpu.make_async_copy(
            rows_v.at[b], out_hbm.at[pl.ds((cc0 + ci) * CH, CH), :], ssem.at[b])

    for a in range(GA):
        g_copy(a, a).start()

    def outer(g, carry):
        for b in range(NB):
            ci = g * NB + b
            nb = (b + GA) % NB

            @pl.when(ci + GA < CHUNKS)
            def _():
                @pl.when(ci >= NB - GA)
                def _():
                    s_copy(ci + GA - NB, nb).wait()
                g_copy(ci + GA, nb).start()

            g_copy(ci, b).wait()
            p_loc = (cc0 + ci) // CPP - ab
            pv = [pos_v[p_loc, pl.ds(c * LANES, LANES)] for c in range(D // LANES)]

            def row(rr, c2):
                for k in range(RU):
                    r = rr * RU + k
                    for c in range(D // LANES):
                        sl = pl.ds(c * LANES, LANES)
                        rows_v[b, r, sl] = rows_v[b, r, sl] + pv[c]
                return c2

            lax.fori_loop(0, CH // RU, row, 0)
            s_copy(ci, b).start()
        return carry

    lax.fori_loop(0, CHUNKS // NB, outer, 0)
    for b in range(NB):
        s_copy(CHUNKS - NB + b, b).wait()


@jax.jit
def kernel(input_ids, weight, position_embedding):
    mesh = plsc.VectorSubcoreMesh(core_axis_name="c", subcore_axis_name="s")
    ids_t = input_ids.astype(jnp.int32).T.reshape(NW, CHUNKS, CH)
    pos_pad = jnp.pad(position_embedding, ((0, PW - S % 8), (0, 0)))
    out = pl.kernel(
        _emb_body,
        mesh=mesh,
        out_type=jax.ShapeDtypeStruct((S * B, D), jnp.float32),
        scratch_types=[
            pltpu.VMEM((CHUNKS, CH), jnp.int32),
            pltpu.VMEM((NB, CH, D), jnp.float32),
            pltpu.VMEM((PW, D), jnp.float32),
            pltpu.SemaphoreType.DMA((NB,)),
            pltpu.SemaphoreType.DMA((NB,)),
        ],
    )(ids_t, weight, pos_pad)
    return out.reshape(S, B, D).transpose(1, 0, 2)


# speedup vs baseline: 5.2279x; 1.0027x over previous
import jax
import jax.numpy as jnp
from jax import lax
from jax.experimental import pallas as pl
from jax.experimental.pallas import tpu as pltpu
from jax.experimental.pallas import tpu_sc as plsc

B, S, D = 4096, 77, 128
LANES = 16
CH = 128
NB = 7
GA = 3
RU = 8
PW = 16

_info = plsc.get_sparse_core_info()
NC, NS = _info.num_cores, _info.num_subcores
NW = NC * NS
ROWS_PER_W = S * B // NW
CHUNKS = ROWS_PER_W // CH
CPP = B // CH


def _emb_body(ids_hbm, w_hbm, pos_hbm, out_hbm, idx_v, rows_v, pos_v, gsem, ssem):
    wid = lax.axis_index("s") * NC + lax.axis_index("c")
    cc0 = wid * CHUNKS
    p_lo = cc0 // CPP
    ab = pl.multiple_of((p_lo // 8) * 8, 8)
    pltpu.sync_copy(pos_hbm.at[pl.ds(ab, PW), :], pos_v)
    pltpu.sync_copy(ids_hbm.at[wid], idx_v)

    def g_copy(ci, b):
        return pltpu.make_async_copy(
            w_hbm.at[idx_v.at[ci]], rows_v.at[b], gsem.at[b])

    def s_copy(ci, b):
        return pltpu.make_async_copy(
            rows_v.at[b], out_hbm.at[pl.ds((cc0 + ci) * CH, CH), :], ssem.at[b])

    for a in range(GA):
        g_copy(a, a).start()

    def outer(g, carry):
        for b in range(NB):
            ci = g * NB + b
            nb = (b + GA) % NB

            @pl.when(ci + GA < CHUNKS)
            def _():
                @pl.when(ci >= NB - GA)
                def _():
                    s_copy(ci + GA - NB, nb).wait()
                g_copy(ci + GA, nb).start()

            g_copy(ci, b).wait()
            p_loc = (cc0 + ci) // CPP - ab
            pv = [pos_v[p_loc, pl.ds(c * LANES, LANES)] for c in range(D // LANES)]

            def row(rr, c2):
                for k in range(RU):
                    r = rr * RU + k
                    for c in range(D // LANES):
                        sl = pl.ds(c * LANES, LANES)
                        rows_v[b, r, sl] = rows_v[b, r, sl] + pv[c]
                return c2

            lax.fori_loop(0, CH // RU, row, 0)
            s_copy(ci, b).start()
        return carry

    lax.fori_loop(0, CHUNKS // NB, outer, 0)
    for b in range(NB):
        s_copy(CHUNKS - NB + b, b).wait()


@jax.jit
def kernel(input_ids, weight, position_embedding):
    mesh = plsc.VectorSubcoreMesh(core_axis_name="c", subcore_axis_name="s")
    ids_t = input_ids.astype(jnp.int32).T.reshape(NW, CHUNKS, CH)
    pos_pad = jnp.pad(position_embedding, ((0, PW - S % 8), (0, 0)))
    out = pl.kernel(
        _emb_body,
        mesh=mesh,
        out_type=jax.ShapeDtypeStruct((S * B, D), jnp.float32),
        scratch_types=[
            pltpu.VMEM((CHUNKS, CH), jnp.int32),
            pltpu.VMEM((NB, CH, D), jnp.float32),
            pltpu.VMEM((PW, D), jnp.float32),
            pltpu.SemaphoreType.DMA((NB,)),
            pltpu.SemaphoreType.DMA((NB,)),
        ],
    )(ids_t, weight, pos_pad)
    return out.reshape(S, B, D).transpose(1, 0, 2)
